# Initial kernel scaffold; baseline (speedup 1.0000x reference)
import jax
import jax.numpy as jnp
from jax.experimental import pallas as pl


def kernel(x, edge_index, edge_attr, batch, We, be, eps, W1, b1, W2, b2, ln_g, ln_b, Wg, bg, Wr1, br1, g1, beta1, Wr2, br2, g2, beta2, Wo, bo):
    raise NotImplementedError("write your pallas kernel here")


# speedup vs baseline: 1.2438x; 1.2438x over previous
import functools

import jax
import jax.numpy as jnp
from jax import lax
from jax.experimental import pallas as pl
from jax.experimental.pallas import tpu as pltpu
from jax.experimental.pallas import tpu_sc as plsc

N = 10000
E = 320000
D = 128
ED = 16
L = 4
G = 256

NC = 2
NS = 16
LANES = 16
NW = NC * NS
EPW = E // NW
C = 80
NCHUNK = EPW // C
NCHUNK_SC = (E // NC) // C
NPAD = 10240
ROWS_PER_TILE = NPAD // NS


def _edge_mm_body(ea_ref, w_ref, b_ref, out_ref):
    out_ref[...] = (
        jnp.dot(ea_ref[...], w_ref[...], preferred_element_type=jnp.float32)
        + b_ref[...]
    )


def _edge_mm(edge_attr, W, b):
    EB = 4000
    return pl.pallas_call(
        _edge_mm_body,
        grid=(E // EB,),
        in_specs=[
            pl.BlockSpec((EB, ED), lambda i: (i, 0)),
            pl.BlockSpec((ED, D), lambda i: (0, 0)),
            pl.BlockSpec((1, D), lambda i: (0, 0)),
        ],
        out_specs=pl.BlockSpec((EB, D), lambda i: (i, 0)),
        out_shape=jax.ShapeDtypeStruct((E, D), jnp.float32),
    )(edge_attr, W, b.reshape(1, D))


GARB = NPAD - 8


def _sc_edge_body(h_hbm, src_hbm, dst_hbm, pdst_hbm, e_hbm, z_hbm, out_hbm,
                  srcv, dstv, pdstv, idxv, flagv, gath, ebuf, stg, tail, tidx,
                  accbuf, agg_sh, sem):
    c = lax.axis_index("c")
    s = lax.axis_index("s")
    wstart = (126 * jnp.minimum(s, 11)
              + 123 * jnp.clip(s - 11, 0, 4)) + c * (NCHUNK_SC)
    wn = jnp.where(s < 11, 126, jnp.where(s < 15, 123, 122))

    pltpu.sync_copy(z_hbm.at[pl.ds(s * ROWS_PER_TILE, ROWS_PER_TILE)],
                    agg_sh.at[pl.ds(s * ROWS_PER_TILE, ROWS_PER_TILE)])
    plsc.subcore_barrier()

    zero = jnp.zeros((LANES,), jnp.float32)
    garbv = jnp.full((LANES,), GARB, jnp.int32)
    onev = jnp.full((LANES,), 1, jnp.int32)
    zeroi = jnp.full((LANES,), 0, jnp.int32)
    NV = D // LANES
    for j in range(NV):
        accbuf[pl.ds(j * LANES, LANES)] = zero

    def chunk(i, carry):
        base = (wstart + i) * C
        pltpu.sync_copy(src_hbm.at[pl.ds(base, C)], srcv)
        pltpu.sync_copy(dst_hbm.at[pl.ds(base, C)], dstv)
        pltpu.sync_copy(pdst_hbm.at[pl.ds(base, C)], pdstv)
        pltpu.async_copy(h_hbm.at[srcv], gath, sem).wait()
        pltpu.sync_copy(e_hbm.at[pl.ds(base, C)], ebuf)

        for g in range(C // LANES):
            sl = pl.ds(g * LANES, LANES)
            ne = dstv[sl] != pdstv[sl]
            idxv[sl] = jnp.where(ne, pdstv[sl], garbv)
            flagv[sl] = jnp.where(ne, onev, zeroi)

        def grp(g, car):
            fv = flagv[pl.ds(g * LANES, LANES)]
            for lane in range(LANES):
                r = g * LANES + lane
                is_new = fv[lane] != 0
                for j in range(NV):
                    sl = pl.ds(j * LANES, LANES)
                    a = accbuf[sl]
                    stg[r, sl] = a
                    mv = jnp.maximum(gath[r, sl] + ebuf[r, sl], 0.0)
                    accbuf[sl] = jnp.where(is_new, mv, a + mv)
            return car

        lax.fori_loop(0, C // LANES, grp, 0)
        pltpu.sync_copy(stg, agg_sh.at[idxv], add=True)
        return carry

    lax.fori_loop(0, wn, chunk, 0)

    lastv = dstv[pl.ds(C - LANES, LANES)]
    tidx[...] = jnp.where(lax.iota(jnp.int32, LANES) == LANES - 1,
                          lastv, garbv)
    for j in range(NV):
        sl = pl.ds(j * LANES, LANES)
        tail[LANES - 1, sl] = accbuf[sl]
        for k in range(LANES - 1):
            tail[k, sl] = zero
    pltpu.sync_copy(tail, agg_sh.at[tidx], add=True)
    plsc.subcore_barrier()

    pltpu.sync_copy(agg_sh.at[pl.ds(s * ROWS_PER_TILE, ROWS_PER_TILE)],
                    out_hbm.at[c, pl.ds(s * ROWS_PER_TILE, ROWS_PER_TILE)])


@functools.cache
def _sc_edge_call():
    return functools.partial(
        pl.kernel,
        out_type=jax.ShapeDtypeStruct((NC, NPAD, D), jnp.float32),
        mesh=plsc.VectorSubcoreMesh(core_axis_name="c", subcore_axis_name="s",
                                    num_cores=NC, num_subcores=NS),
        scratch_types=[
            pltpu.VMEM((C,), jnp.int32),
            pltpu.VMEM((C,), jnp.int32),
            pltpu.VMEM((C,), jnp.int32),
            pltpu.VMEM((C,), jnp.int32),
            pltpu.VMEM((C,), jnp.int32),
            pltpu.VMEM((C, D), jnp.float32),
            pltpu.VMEM((C, D), jnp.float32),
            pltpu.VMEM((C, D), jnp.float32),
            pltpu.VMEM((LANES, D), jnp.float32),
            pltpu.VMEM((LANES,), jnp.int32),
            pltpu.VMEM((D,), jnp.float32),
            pltpu.VMEM_SHARED((NPAD, D), jnp.float32),
            pltpu.SemaphoreType.DMA,
        ],
    )(_sc_edge_body)


def _sc_edge(h, src, dst, pdst, e, zeros):
    return _sc_edge_call()(h, src, dst, pdst, e, zeros)


def _dense_body(h_ref, p_ref, epsv_ref, w1_ref, b1_ref, w2_ref, b2_ref,
                g_ref, bln_ref, out_ref):
    h = h_ref[...]
    z = epsv_ref[...] * h + (p_ref[0] + p_ref[1])
    z = jnp.dot(z, w1_ref[...], preferred_element_type=jnp.float32) + b1_ref[...]
    z = jnp.maximum(z, 0.0)
    z = jnp.dot(z, w2_ref[...], preferred_element_type=jnp.float32) + b2_ref[...]
    mu = jnp.mean(z, axis=-1, keepdims=True)
    zc = z - mu
    var = jnp.mean(zc * zc, axis=-1, keepdims=True)
    z = zc * lax.rsqrt(var + 1e-5) * g_ref[...] + bln_ref[...]
    out_ref[...] = jnp.maximum(z + h, 0.0)


def _dense(h, parts, epsv, W1, b1, W2, b2, ln_g, ln_b):
    NB = 1000
    return pl.pallas_call(
        _dense_body,
        grid=(N // NB,),
        in_specs=[
            pl.BlockSpec((NB, D), lambda i: (i, 0)),
            pl.BlockSpec((NC, NB, D), lambda i: (0, i, 0)),
            pl.BlockSpec((1, 1), lambda i: (0, 0)),
            pl.BlockSpec((D, D), lambda i: (0, 0)),
            pl.BlockSpec((1, D), lambda i: (0, 0)),
            pl.BlockSpec((D, D), lambda i: (0, 0)),
            pl.BlockSpec((1, D), lambda i: (0, 0)),
            pl.BlockSpec((1, D), lambda i: (0, 0)),
            pl.BlockSpec((1, D), lambda i: (0, 0)),
        ],
        out_specs=pl.BlockSpec((NB, D), lambda i: (i, 0)),
        out_shape=jax.ShapeDtypeStruct((N, D), jnp.float32),
    )(h, parts, epsv, W1, b1.reshape(1, D), W2, b2.reshape(1, D),
      ln_g.reshape(1, D), ln_b.reshape(1, D))


def _gate_body(h_ref, wg_ref, bg_ref, gate_ref, gmax_ref, m_ref):
    i = pl.program_id(0)
    g = (jnp.dot(h_ref[...], wg_ref[...], preferred_element_type=jnp.float32)
         + bg_ref[...])
    gate_ref[...] = g
    bm = jnp.max(g)

    @pl.when(i == 0)
    def _():
        m_ref[0, 0] = bm

    @pl.when(i > 0)
    def _():
        m_ref[0, 0] = jnp.maximum(m_ref[0, 0], bm)

    gmax_ref[...] = jnp.broadcast_to(m_ref[0, 0], (1, 1))


def _gate(h, Wg, bg):
    NB = 1000
    return pl.pallas_call(
        _gate_body,
        grid=(N // NB,),
        in_specs=[
            pl.BlockSpec((NB, D), lambda i: (i, 0)),
            pl.BlockSpec((D, 1), lambda i: (0, 0)),
            pl.BlockSpec((1, 1), lambda i: (0, 0)),
        ],
        out_specs=[
            pl.BlockSpec((NB, 1), lambda i: (i, 0)),
            pl.BlockSpec((1, 1), lambda i: (0, 0)),
        ],
        out_shape=[
            jax.ShapeDtypeStruct((N, 1), jnp.float32),
            jax.ShapeDtypeStruct((1, 1), jnp.float32),
        ],
        scratch_shapes=[pltpu.SMEM((1, 1), jnp.float32)],
    )(h, Wg, bg.reshape(1, 1))


def _pool_body(h_ref, gate_ref, gmax_ref, batch_ref,
               wr1_ref, br1_ref, g1_ref, be1_ref,
               wr2_ref, br2_ref, g2_ref, be2_ref,
               wo_ref, bo_ref, out_ref, accp_ref, accd_ref):
    i = pl.program_id(0)
    nb = pl.num_programs(0)
    ex = jnp.exp(gate_ref[...] - gmax_ref[...])
    onehot = (batch_ref[...] ==
              lax.broadcasted_iota(jnp.int32, (1, G), 1)).astype(jnp.float32)
    exh = ex * h_ref[...]
    exb = jnp.broadcast_to(ex, exh.shape)
    dn = (((0,), (0,)), ((), ()))
    p = lax.dot_general(onehot, exh, dn, precision=lax.Precision.HIGHEST,
                        preferred_element_type=jnp.float32)
    d = lax.dot_general(onehot, exb, dn, precision=lax.Precision.HIGHEST,
                        preferred_element_type=jnp.float32)

    @pl.when(i == 0)
    def _():
        accp_ref[...] = p
        accd_ref[...] = d

    @pl.when(i > 0)
    def _():
        accp_ref[...] += p
        accd_ref[...] += d

    @pl.when(i == nb - 1)
    def _():
        pooled = accp_ref[...] / (accd_ref[...] + 1e-12)
        r = (jnp.dot(pooled, wr1_ref[...], preferred_element_type=jnp.float32)
             + br1_ref[...])
        mu = jnp.mean(r, axis=0, keepdims=True)
        rc = r - mu
        var = jnp.mean(rc * rc, axis=0, keepdims=True)
        r = jnp.maximum(rc * lax.rsqrt(var + 1e-5) * g1_ref[...]
                        + be1_ref[...], 0.0)
        r = (jnp.dot(r, wr2_ref[...], preferred_element_type=jnp.float32)
             + br2_ref[...])
        mu = jnp.mean(r, axis=0, keepdims=True)
        rc = r - mu
        var = jnp.mean(rc * rc, axis=0, keepdims=True)
        r = jnp.maximum(rc * lax.rsqrt(var + 1e-5) * g2_ref[...]
                        + be2_ref[...], 0.0)
        out_ref[...] = (jnp.dot(r, wo_ref[...],
                                preferred_element_type=jnp.float32)
                        + bo_ref[...])


def _pool_regress(h, gate, gmax, batch2d, Wr1, br1, g1, beta1,
                  Wr2p, br2p, g2p, beta2p, Wop, bo):
    NB = 1000
    return pl.pallas_call(
        _pool_body,
        grid=(N // NB,),
        in_specs=[
            pl.BlockSpec((NB, D), lambda i: (i, 0)),
            pl.BlockSpec((NB, 1), lambda i: (i, 0)),
            pl.BlockSpec((1, 1), lambda i: (0, 0)),
            pl.BlockSpec((NB, 1), lambda i: (i, 0)),
            pl.BlockSpec((D, D), lambda i: (0, 0)),
            pl.BlockSpec((1, D), lambda i: (0, 0)),
            pl.BlockSpec((1, D), lambda i: (0, 0)),
            pl.BlockSpec((1, D), lambda i: (0, 0)),
            pl.BlockSpec((D, D), lambda i: (0, 0)),
            pl.BlockSpec((1, D), lambda i: (0, 0)),
            pl.BlockSpec((1, D), lambda i: (0, 0)),
            pl.BlockSpec((1, D), lambda i: (0, 0)),
            pl.BlockSpec((D, 1), lambda i: (0, 0)),
            pl.BlockSpec((1, 1), lambda i: (0, 0)),
        ],
        out_specs=pl.BlockSpec((G, 1), lambda i: (0, 0)),
        out_shape=jax.ShapeDtypeStruct((G, 1), jnp.float32),
        scratch_shapes=[
            pltpu.VMEM((G, D), jnp.float32),
            pltpu.VMEM((G, D), jnp.float32),
        ],
    )(h, gate, gmax, batch2d, Wr1, br1.reshape(1, D), g1.reshape(1, D),
      beta1.reshape(1, D), Wr2p, br2p, g2p, beta2p, Wop, bo.reshape(1, 1))


def kernel(x, edge_index, edge_attr, batch, We, be, eps, W1, b1, W2, b2,
           ln_g, ln_b, Wg, bg, Wr1, br1, g1, beta1, Wr2, br2, g2, beta2,
           Wo, bo):
    perm = jnp.argsort(edge_index[1], stable=True)
    src = edge_index[0][perm]
    dst = edge_index[1][perm]
    edge_attr = edge_attr[perm]
    pdst = jnp.concatenate(
        [jnp.full((1,), GARB, jnp.int32), dst[:-1]])
    zeros = jnp.zeros((NPAD, D), jnp.float32)

    h = x
    for l in range(L):
        e = _edge_mm(edge_attr, We[l], be[l])
        parts = _sc_edge(h, src, dst, pdst, e, zeros)
        epsv = (1.0 + eps[l]).reshape(1, 1)
        h = _dense(h, parts, epsv, W1[l], b1[l], W2[l], b2[l],
                   ln_g[l], ln_b[l])

    gate, gmax = _gate(h, Wg, bg)
    batch2d = batch.reshape(N, 1)

    H2 = 64
    Wr2p = jnp.pad(Wr2, ((0, 0), (0, D - H2)))
    br2p = jnp.pad(br2, (0, D - H2)).reshape(1, D)
    g2p = jnp.pad(g2, (0, D - H2), constant_values=1.0).reshape(1, D)
    beta2p = jnp.pad(beta2, (0, D - H2)).reshape(1, D)
    Wop = jnp.pad(Wo, ((0, D - H2), (0, 0)))

    return _pool_regress(h, gate, gmax, batch2d, Wr1, br1, g1, beta1,
                         Wr2p, br2p, g2p, beta2p, Wop, bo)
